# baseline (speedup 1.0000x reference)
import functools

import jax
import jax.numpy as jnp
import numpy as np
from jax.experimental import pallas as pl
from jax.experimental.pallas import tpu as pltpu

_TOP_K = 16


def _bdot(a, b):
    return jax.lax.dot_general(a.astype(jnp.bfloat16), b.astype(jnp.bfloat16),
                               (((1,), (1,)), ((), ())),
                               preferred_element_type=jnp.float32)


def _kv_body(side_ref, rel_ref, kw_ref, kb_ref, vw_ref, vb_ref, k_out, v_out):
    side = side_ref[...]
    kin = side * rel_ref[...]
    k_out[...] = _bdot(kin, kw_ref[...]) + kb_ref[...]
    v_out[...] = _bdot(side, vw_ref[...]) + vb_ref[...]


def _select_threshold_flat(s):
    m = jnp.max(s, axis=1, keepdims=True)
    t = m
    for _ in range(_TOP_K - 1):
        t = jnp.max(jnp.where(s < t, s, -jnp.inf), axis=1, keepdims=True)
    return m, t


def _select_threshold_chunked(s, n_side):
    g = n_side // _TOP_K
    neg = jnp.float32(-1e30)
    sl = [s[:, i * g:(i + 1) * g] for i in range(_TOP_K)]
    cm1 = sl[0]
    for x in sl[1:]:
        cm1 = jnp.maximum(cm1, x)
    cm2 = neg * jnp.ones_like(cm1)
    for x in sl:
        cm2 = jnp.maximum(cm2, jnp.where(x == cm1, neg, x))
    cm3 = neg * jnp.ones_like(cm1)
    for x in sl:
        cm3 = jnp.maximum(cm3, jnp.where((x == cm1) | (x == cm2), neg, x))
    w = cm1
    lvl = jnp.zeros(cm1.shape, jnp.int32)
    m = None
    t = None
    for i in range(_TOP_K):
        t = jnp.max(w, axis=1, keepdims=True)
        if i == 0:
            m = t
        if i < _TOP_K - 1:
            sel = w == t
            lvl = lvl + sel.astype(jnp.int32)
            nxt = jnp.where(lvl == 1, cm2, jnp.where(lvl == 2, cm3, neg))
            w = jnp.where(sel, nxt, w)
    return m, t


def _main_body(ego_ref, qw_ref, qb_ref, k_ref, v_ref, out_ref, *, scale, n_side):
    q = _bdot(ego_ref[...], qw_ref[...]) + qb_ref[...]
    s = _bdot(q, k_ref[...]) / scale
    if n_side % _TOP_K == 0 and n_side >= 32 * _TOP_K:
        m, t = _select_threshold_chunked(s, n_side)
    else:
        m, t = _select_threshold_flat(s)
    e = jnp.where(s >= t, jnp.exp(s - m), 0.0)
    denom = jnp.sum(e, axis=1, keepdims=True)
    agg = jax.lax.dot_general(e, v_ref[...], (((1,), (0,)), ((), ())),
                              preferred_element_type=jnp.float32,
                              precision=jax.lax.Precision.HIGHEST)
    out_ref[...] = agg / denom


def _build(n_ego, n_side, d, r_block, kv_block):
    scale = np.float32(np.sqrt(d))
    kv_grid = n_side // kv_block
    kv = pl.pallas_call(
        _kv_body,
        grid=(kv_grid,),
        in_specs=[
            pl.BlockSpec((kv_block, d), lambda i: (i, 0)),
            pl.BlockSpec((kv_block, d), lambda i: (i, 0)),
            pl.BlockSpec((d, d), lambda i: (0, 0)),
            pl.BlockSpec((1, d), lambda i: (0, 0)),
            pl.BlockSpec((d, d), lambda i: (0, 0)),
            pl.BlockSpec((1, d), lambda i: (0, 0)),
        ],
        out_specs=[
            pl.BlockSpec((kv_block, d), lambda i: (i, 0)),
            pl.BlockSpec((kv_block, d), lambda i: (i, 0)),
        ],
        out_shape=[
            jax.ShapeDtypeStruct((n_side, d), jnp.float32),
            jax.ShapeDtypeStruct((n_side, d), jnp.float32),
        ],
    )
    main_grid = n_ego // r_block
    main = pl.pallas_call(
        functools.partial(_main_body, scale=scale, n_side=n_side),
        grid=(main_grid,),
        in_specs=[
            pl.BlockSpec((r_block, d), lambda i: (i, 0)),
            pl.BlockSpec((d, d), lambda i: (0, 0)),
            pl.BlockSpec((1, d), lambda i: (0, 0)),
            pl.BlockSpec((n_side, d), lambda i: (0, 0)),
            pl.BlockSpec((n_side, d), lambda i: (0, 0)),
        ],
        out_specs=pl.BlockSpec((r_block, d), lambda i: (i, 0)),
        out_shape=jax.ShapeDtypeStruct((n_ego, d), jnp.float32),
    )
    return kv, main


def kernel(ego_emb, side_emb, rel_emb, q_w, q_b, k_w, k_b, v_w, v_b):
    n_ego, d = ego_emb.shape
    n_side = side_emb.shape[0]
    r_block = 400 if n_ego % 400 == 0 else n_ego
    kv_block = 2000 if n_side % 2000 == 0 else n_side
    kv, main = _build(n_ego, n_side, d, r_block, kv_block)
    k_mat, v_mat = kv(side_emb, rel_emb, k_w, k_b.reshape(1, d),
                      v_w, v_b.reshape(1, d))
    return main(ego_emb, q_w, q_b.reshape(1, d), k_mat, v_mat)


# speedup vs baseline: 1.8954x; 1.8954x over previous
import functools

import jax
import jax.numpy as jnp
import numpy as np
from jax.experimental import pallas as pl
from jax.experimental.pallas import tpu as pltpu

_TOP_K = 16


def _bdot(a, b):
    return jax.lax.dot_general(a.astype(jnp.bfloat16), b.astype(jnp.bfloat16),
                               (((1,), (1,)), ((), ())),
                               preferred_element_type=jnp.float32)


def _kv_body(side_ref, rel_ref, kw_ref, kb_ref, vw_ref, vb_ref, k_out, v_out):
    side = side_ref[...]
    kin = side * rel_ref[...]
    k_out[...] = _bdot(kin, kw_ref[...]) + kb_ref[...]
    v_out[...] = _bdot(side, vw_ref[...]) + vb_ref[...]


def _select_threshold_flat(s):
    m = jnp.max(s, axis=1, keepdims=True)
    t = m
    for _ in range(_TOP_K - 1):
        t = jnp.max(jnp.where(s < t, s, -jnp.inf), axis=1, keepdims=True)
    return m, t


def _select_threshold_chunked(s, n_side):
    g = n_side // _TOP_K
    neg = jnp.float32(-1e30)
    sl = [s[:, i * g:(i + 1) * g] for i in range(_TOP_K)]
    cm1 = sl[0]
    for x in sl[1:]:
        cm1 = jnp.maximum(cm1, x)
    cm2 = neg * jnp.ones_like(cm1)
    for x in sl:
        cm2 = jnp.maximum(cm2, jnp.where(x == cm1, neg, x))
    cm3 = neg * jnp.ones_like(cm1)
    for x in sl:
        cm3 = jnp.maximum(cm3, jnp.where((x == cm1) | (x == cm2), neg, x))
    w = cm1
    lvl = jnp.zeros(cm1.shape, jnp.int32)
    m = None
    t = None
    for i in range(_TOP_K):
        t = jnp.max(w, axis=1, keepdims=True)
        if i == 0:
            m = t
        if i < _TOP_K - 1:
            sel = w == t
            lvl = lvl + sel.astype(jnp.int32)
            nxt = jnp.where(lvl == 1, cm2, jnp.where(lvl == 2, cm3, neg))
            w = jnp.where(sel, nxt, w)
    return m, t


def _main_body(ego_ref, qw_ref, qb_ref, k_ref, v_ref, out_ref, *, scale, n_side):
    q = _bdot(ego_ref[...], qw_ref[...]) + qb_ref[...]
    s = _bdot(q, k_ref[...]) / scale
    if n_side % _TOP_K == 0 and n_side >= 32 * _TOP_K:
        m, t = _select_threshold_chunked(s, n_side)
    else:
        m, t = _select_threshold_flat(s)
    out_ref[...] = t + m + v_ref[0:1, :] * 0.0


def _build(n_ego, n_side, d, r_block, kv_block):
    scale = np.float32(np.sqrt(d))
    kv_grid = n_side // kv_block
    kv = pl.pallas_call(
        _kv_body,
        grid=(kv_grid,),
        in_specs=[
            pl.BlockSpec((kv_block, d), lambda i: (i, 0)),
            pl.BlockSpec((kv_block, d), lambda i: (i, 0)),
            pl.BlockSpec((d, d), lambda i: (0, 0)),
            pl.BlockSpec((1, d), lambda i: (0, 0)),
            pl.BlockSpec((d, d), lambda i: (0, 0)),
            pl.BlockSpec((1, d), lambda i: (0, 0)),
        ],
        out_specs=[
            pl.BlockSpec((kv_block, d), lambda i: (i, 0)),
            pl.BlockSpec((kv_block, d), lambda i: (i, 0)),
        ],
        out_shape=[
            jax.ShapeDtypeStruct((n_side, d), jnp.float32),
            jax.ShapeDtypeStruct((n_side, d), jnp.float32),
        ],
    )
    main_grid = n_ego // r_block
    main = pl.pallas_call(
        functools.partial(_main_body, scale=scale, n_side=n_side),
        grid=(main_grid,),
        in_specs=[
            pl.BlockSpec((r_block, d), lambda i: (i, 0)),
            pl.BlockSpec((d, d), lambda i: (0, 0)),
            pl.BlockSpec((1, d), lambda i: (0, 0)),
            pl.BlockSpec((n_side, d), lambda i: (0, 0)),
            pl.BlockSpec((n_side, d), lambda i: (0, 0)),
        ],
        out_specs=pl.BlockSpec((r_block, d), lambda i: (i, 0)),
        out_shape=jax.ShapeDtypeStruct((n_ego, d), jnp.float32),
    )
    return kv, main


def kernel(ego_emb, side_emb, rel_emb, q_w, q_b, k_w, k_b, v_w, v_b):
    n_ego, d = ego_emb.shape
    n_side = side_emb.shape[0]
    r_block = 400 if n_ego % 400 == 0 else n_ego
    kv_block = 2000 if n_side % 2000 == 0 else n_side
    kv, main = _build(n_ego, n_side, d, r_block, kv_block)
    k_mat, v_mat = kv(side_emb, rel_emb, k_w, k_b.reshape(1, d),
                      v_w, v_b.reshape(1, d))
    return main(ego_emb, q_w, q_b.reshape(1, d), k_mat, v_mat)


# speedup vs baseline: 2.2864x; 1.2063x over previous
import functools

import jax
import jax.numpy as jnp
import numpy as np
from jax.experimental import pallas as pl
from jax.experimental.pallas import tpu as pltpu

_TOP_K = 16


def _bdot(a, b):
    return jax.lax.dot_general(a.astype(jnp.bfloat16), b.astype(jnp.bfloat16),
                               (((1,), (1,)), ((), ())),
                               preferred_element_type=jnp.float32)


def _kv_body(side_ref, rel_ref, kw_ref, kb_ref, vw_ref, vb_ref, k_out, v_out):
    side = side_ref[...]
    kin = side * rel_ref[...]
    k_out[...] = _bdot(kin, kw_ref[...]) + kb_ref[...]
    v_out[...] = _bdot(side, vw_ref[...]) + vb_ref[...]


def _select_threshold_flat(s):
    m = jnp.max(s, axis=1, keepdims=True)
    t = m
    for _ in range(_TOP_K - 1):
        t = jnp.max(jnp.where(s < t, s, -jnp.inf), axis=1, keepdims=True)
    return m, t


def _select_threshold_chunked(s, n_side):
    g = n_side // _TOP_K
    neg = jnp.float32(-1e30)
    sl = [s[:, i * g:(i + 1) * g] for i in range(_TOP_K)]
    cm1 = sl[0]
    for x in sl[1:]:
        cm1 = jnp.maximum(cm1, x)
    cm2 = neg * jnp.ones_like(cm1)
    for x in sl:
        cm2 = jnp.maximum(cm2, jnp.where(x == cm1, neg, x))
    cm3 = neg * jnp.ones_like(cm1)
    for x in sl:
        cm3 = jnp.maximum(cm3, jnp.where((x == cm1) | (x == cm2), neg, x))
    w = cm1
    lvl = jnp.zeros(cm1.shape, jnp.int32)
    m = None
    t = None
    for i in range(_TOP_K):
        t = jnp.max(w, axis=1, keepdims=True)
        if i == 0:
            m = t
        if i < _TOP_K - 1:
            sel = w == t
            lvl = lvl + sel.astype(jnp.int32)
            nxt = jnp.where(lvl == 1, cm2, jnp.where(lvl == 2, cm3, neg))
            w = jnp.where(sel, nxt, w)
    return m, t


def _main_body(ego_ref, qw_ref, qb_ref, k_ref, v_ref, out_ref, *, scale, n_side):
    q = _bdot(ego_ref[...], qw_ref[...]) + qb_ref[...]
    s = _bdot(q, k_ref[...]) / scale
    g = n_side // _TOP_K
    neg = jnp.float32(-1e30)
    sl = [s[:, i * g:(i + 1) * g] for i in range(_TOP_K)]
    cm1 = sl[0]
    for x in sl[1:]:
        cm1 = jnp.maximum(cm1, x)
    cm2 = neg * jnp.ones_like(cm1)
    for x in sl:
        cm2 = jnp.maximum(cm2, jnp.where(x == cm1, neg, x))
    cm3 = neg * jnp.ones_like(cm1)
    for x in sl:
        cm3 = jnp.maximum(cm3, jnp.where((x == cm1) | (x == cm2), neg, x))
    t = jnp.max(cm3, axis=1, keepdims=True)
    out_ref[...] = t + v_ref[0:1, :] * 0.0


def _build(n_ego, n_side, d, r_block, kv_block):
    scale = np.float32(np.sqrt(d))
    kv_grid = n_side // kv_block
    kv = pl.pallas_call(
        _kv_body,
        grid=(kv_grid,),
        in_specs=[
            pl.BlockSpec((kv_block, d), lambda i: (i, 0)),
            pl.BlockSpec((kv_block, d), lambda i: (i, 0)),
            pl.BlockSpec((d, d), lambda i: (0, 0)),
            pl.BlockSpec((1, d), lambda i: (0, 0)),
            pl.BlockSpec((d, d), lambda i: (0, 0)),
            pl.BlockSpec((1, d), lambda i: (0, 0)),
        ],
        out_specs=[
            pl.BlockSpec((kv_block, d), lambda i: (i, 0)),
            pl.BlockSpec((kv_block, d), lambda i: (i, 0)),
        ],
        out_shape=[
            jax.ShapeDtypeStruct((n_side, d), jnp.float32),
            jax.ShapeDtypeStruct((n_side, d), jnp.float32),
        ],
    )
    main_grid = n_ego // r_block
    main = pl.pallas_call(
        functools.partial(_main_body, scale=scale, n_side=n_side),
        grid=(main_grid,),
        in_specs=[
            pl.BlockSpec((r_block, d), lambda i: (i, 0)),
            pl.BlockSpec((d, d), lambda i: (0, 0)),
            pl.BlockSpec((1, d), lambda i: (0, 0)),
            pl.BlockSpec((n_side, d), lambda i: (0, 0)),
            pl.BlockSpec((n_side, d), lambda i: (0, 0)),
        ],
        out_specs=pl.BlockSpec((r_block, d), lambda i: (i, 0)),
        out_shape=jax.ShapeDtypeStruct((n_ego, d), jnp.float32),
    )
    return kv, main


def kernel(ego_emb, side_emb, rel_emb, q_w, q_b, k_w, k_b, v_w, v_b):
    n_ego, d = ego_emb.shape
    n_side = side_emb.shape[0]
    r_block = 400 if n_ego % 400 == 0 else n_ego
    kv_block = 2000 if n_side % 2000 == 0 else n_side
    kv, main = _build(n_ego, n_side, d, r_block, kv_block)
    k_mat, v_mat = kv(side_emb, rel_emb, k_w, k_b.reshape(1, d),
                      v_w, v_b.reshape(1, d))
    return main(ego_emb, q_w, q_b.reshape(1, d), k_mat, v_mat)


# speedup vs baseline: 2.7363x; 1.1968x over previous
import functools

import jax
import jax.numpy as jnp
import numpy as np
from jax.experimental import pallas as pl
from jax.experimental.pallas import tpu as pltpu

_TOP_K = 16


def _bdot(a, b):
    return jax.lax.dot_general(a.astype(jnp.bfloat16), b.astype(jnp.bfloat16),
                               (((1,), (1,)), ((), ())),
                               preferred_element_type=jnp.float32)


def _kv_body(side_ref, rel_ref, kw_ref, kb_ref, vw_ref, vb_ref, k_out, v_out):
    side = side_ref[...]
    kin = side * rel_ref[...]
    k_out[...] = _bdot(kin, kw_ref[...]) + kb_ref[...]
    v_out[...] = _bdot(side, vw_ref[...]) + vb_ref[...]


def _select_threshold_flat(s):
    m = jnp.max(s, axis=1, keepdims=True)
    t = m
    for _ in range(_TOP_K - 1):
        t = jnp.max(jnp.where(s < t, s, -jnp.inf), axis=1, keepdims=True)
    return m, t


def _select_threshold_aligned(s, n_side):
    p = 128
    nfull = n_side // p
    tail_w = n_side - nfull * p
    neg = jnp.float32(-1e30)
    sl = [s[:, i * p:(i + 1) * p] for i in range(nfull)]
    cm1 = sl[0]
    for x in sl[1:]:
        cm1 = jnp.maximum(cm1, x)
    cm2 = neg * jnp.ones_like(cm1)
    for x in sl:
        cm2 = jnp.maximum(cm2, jnp.where(x == cm1, neg, x))
    cm3 = neg * jnp.ones_like(cm1)
    for x in sl:
        cm3 = jnp.maximum(cm3, jnp.where((x == cm1) | (x == cm2), neg, x))
    cm4 = neg * jnp.ones_like(cm1)
    for x in sl:
        cm4 = jnp.maximum(
            cm4, jnp.where((x == cm1) | (x == cm2) | (x == cm3), neg, x))
    if tail_w:
        tail = s[:, nfull * p:]
        negt = neg * jnp.ones_like(tail)
        w = jnp.concatenate([cm1, tail], axis=1)
        cm2, cm3, cm4 = (jnp.concatenate([c, negt], axis=1)
                         for c in (cm2, cm3, cm4))
    else:
        w = cm1
    lvl = jnp.zeros(w.shape, jnp.int32)
    m = None
    t = None
    for i in range(_TOP_K):
        t = jnp.max(w, axis=1, keepdims=True)
        if i == 0:
            m = t
        if i < _TOP_K - 1:
            sel = w == t
            lvl = lvl + sel.astype(jnp.int32)
            nxt = jnp.where(lvl == 1, cm2,
                            jnp.where(lvl == 2, cm3,
                                      jnp.where(lvl == 3, cm4, neg)))
            w = jnp.where(sel, nxt, w)
    return m, t


def _select_threshold_chunked(s, n_side):
    g = n_side // _TOP_K
    neg = jnp.float32(-1e30)
    sl = [s[:, i * g:(i + 1) * g] for i in range(_TOP_K)]
    cm1 = sl[0]
    for x in sl[1:]:
        cm1 = jnp.maximum(cm1, x)
    cm2 = neg * jnp.ones_like(cm1)
    for x in sl:
        cm2 = jnp.maximum(cm2, jnp.where(x == cm1, neg, x))
    cm3 = neg * jnp.ones_like(cm1)
    for x in sl:
        cm3 = jnp.maximum(cm3, jnp.where((x == cm1) | (x == cm2), neg, x))
    w = cm1
    lvl = jnp.zeros(cm1.shape, jnp.int32)
    m = None
    t = None
    for i in range(_TOP_K):
        t = jnp.max(w, axis=1, keepdims=True)
        if i == 0:
            m = t
        if i < _TOP_K - 1:
            sel = w == t
            lvl = lvl + sel.astype(jnp.int32)
            nxt = jnp.where(lvl == 1, cm2, jnp.where(lvl == 2, cm3, neg))
            w = jnp.where(sel, nxt, w)
    return m, t


def _main_body(ego_ref, qw_ref, qb_ref, k_ref, v_ref, out_ref, *, scale, n_side):
    q = _bdot(ego_ref[...], qw_ref[...]) + qb_ref[...]
    s = _bdot(q, k_ref[...]) / scale
    if n_side >= 16 * 128:
        m, t = _select_threshold_aligned(s, n_side)
    elif n_side % _TOP_K == 0 and n_side >= 32 * _TOP_K:
        m, t = _select_threshold_chunked(s, n_side)
    else:
        m, t = _select_threshold_flat(s)
    e = jnp.where(s >= t, jnp.exp(s - m), 0.0)
    denom = jnp.sum(e, axis=1, keepdims=True)
    agg = jax.lax.dot_general(e.astype(jnp.bfloat16),
                              v_ref[...].astype(jnp.bfloat16),
                              (((1,), (0,)), ((), ())),
                              preferred_element_type=jnp.float32)
    out_ref[...] = agg / denom


def _build(n_ego, n_side, d, r_block, kv_block):
    scale = np.float32(np.sqrt(d))
    kv_grid = n_side // kv_block
    kv = pl.pallas_call(
        _kv_body,
        grid=(kv_grid,),
        in_specs=[
            pl.BlockSpec((kv_block, d), lambda i: (i, 0)),
            pl.BlockSpec((kv_block, d), lambda i: (i, 0)),
            pl.BlockSpec((d, d), lambda i: (0, 0)),
            pl.BlockSpec((1, d), lambda i: (0, 0)),
            pl.BlockSpec((d, d), lambda i: (0, 0)),
            pl.BlockSpec((1, d), lambda i: (0, 0)),
        ],
        out_specs=[
            pl.BlockSpec((kv_block, d), lambda i: (i, 0)),
            pl.BlockSpec((kv_block, d), lambda i: (i, 0)),
        ],
        out_shape=[
            jax.ShapeDtypeStruct((n_side, d), jnp.float32),
            jax.ShapeDtypeStruct((n_side, d), jnp.float32),
        ],
    )
    main_grid = n_ego // r_block
    main = pl.pallas_call(
        functools.partial(_main_body, scale=scale, n_side=n_side),
        grid=(main_grid,),
        in_specs=[
            pl.BlockSpec((r_block, d), lambda i: (i, 0)),
            pl.BlockSpec((d, d), lambda i: (0, 0)),
            pl.BlockSpec((1, d), lambda i: (0, 0)),
            pl.BlockSpec((n_side, d), lambda i: (0, 0)),
            pl.BlockSpec((n_side, d), lambda i: (0, 0)),
        ],
        out_specs=pl.BlockSpec((r_block, d), lambda i: (i, 0)),
        out_shape=jax.ShapeDtypeStruct((n_ego, d), jnp.float32),
    )
    return kv, main


def kernel(ego_emb, side_emb, rel_emb, q_w, q_b, k_w, k_b, v_w, v_b):
    n_ego, d = ego_emb.shape
    n_side = side_emb.shape[0]
    r_block = 400 if n_ego % 400 == 0 else n_ego
    kv_block = 2000 if n_side % 2000 == 0 else n_side
    kv, main = _build(n_ego, n_side, d, r_block, kv_block)
    k_mat, v_mat = kv(side_emb, rel_emb, k_w, k_b.reshape(1, d),
                      v_w, v_b.reshape(1, d))
    return main(ego_emb, q_w, q_b.reshape(1, d), k_mat, v_mat)
